# Initial kernel scaffold; baseline (speedup 1.0000x reference)
import functools

import jax
import jax.numpy as jnp
from jax import lax
from jax.experimental import pallas as pl
from jax.experimental.pallas import tpu as pltpu
from jax.experimental.pallas import tpu_sc as plsc

B = 4096
S = 200
D = 32
NW = 32
ROWS_PER_W = B * S // NW
CHUNK = 800
SEQ_PER_CHUNK = CHUNK // S
NCHUNK = ROWS_PER_W // CHUNK

_mesh = plsc.VectorSubcoreMesh(core_axis_name="c", subcore_axis_name="s")


@functools.partial(
    pl.kernel,
    mesh=_mesh,
    out_type=jax.ShapeDtypeStruct((B * S, D), jnp.float32),
    scratch_types=[
        pltpu.VMEM((CHUNK,), jnp.int32),
        pltpu.VMEM((CHUNK, D), jnp.float32),
        pltpu.VMEM((S, D), jnp.float32),
        pltpu.SemaphoreType.DMA,
    ],
)
def _emb_kernel(x_hbm, table_hbm, pe_hbm, out_hbm, idx_v, rows_v, pe_v, sem):
    cid = lax.axis_index("c")
    sid = lax.axis_index("s")
    wid = sid * 2 + cid
    base = wid * ROWS_PER_W

    pltpu.sync_copy(pe_hbm, pe_v)

    def chunk_body(c, carry):
        off = base + c * CHUNK
        pltpu.sync_copy(x_hbm.at[pl.ds(off, CHUNK)], idx_v)
        pltpu.async_copy(table_hbm.at[idx_v], rows_v, sem).wait()

        def seq_body(k, carry2):
            def row_body(s2, carry3):
                r = k * S + s2
                rows_v[r, pl.ds(0, 16)] = (
                    rows_v[r, pl.ds(0, 16)] + pe_v[s2, pl.ds(0, 16)]
                )
                rows_v[r, pl.ds(16, 16)] = (
                    rows_v[r, pl.ds(16, 16)] + pe_v[s2, pl.ds(16, 16)]
                )
                return carry3

            return lax.fori_loop(0, S, row_body, carry2)

        lax.fori_loop(0, SEQ_PER_CHUNK, seq_body, 0)
        pltpu.sync_copy(rows_v, out_hbm.at[pl.ds(off, CHUNK)])
        return carry

    lax.fori_loop(0, NCHUNK, chunk_body, 0)


def kernel(x, table, pe):
    xf = x.reshape(B * S).astype(jnp.int32)
    pe2 = pe[0, :S, :]
    out = _emb_kernel(xf, table, pe2)
    return out.reshape(B, S, D)


# speedup vs baseline: 4.1395x; 4.1395x over previous
import functools

import jax
import jax.numpy as jnp
from jax import lax
from jax.experimental import pallas as pl
from jax.experimental.pallas import tpu as pltpu
from jax.experimental.pallas import tpu_sc as plsc

B = 4096
S = 200
D = 32
NW = 32
ROWS_PER_W = B * S // NW
CHUNK = 800
SEQ_PER_CHUNK = CHUNK // S
NCHUNK = ROWS_PER_W // CHUNK

_mesh = plsc.VectorSubcoreMesh(core_axis_name="c", subcore_axis_name="s")


@functools.partial(
    pl.kernel,
    mesh=_mesh,
    compiler_params=pltpu.CompilerParams(use_tc_tiling_on_sc=False),
    out_type=jax.ShapeDtypeStruct((B * S, D), jnp.float32),
    scratch_types=[
        pltpu.VMEM((CHUNK,), jnp.int32),
        pltpu.VMEM((CHUNK, D), jnp.float32),
        pltpu.VMEM((S, D), jnp.float32),
        pltpu.SemaphoreType.DMA,
    ],
)
def _emb_kernel(x_hbm, table_hbm, pe_hbm, out_hbm, idx_v, rows_v, pe_v, sem):
    cid = lax.axis_index("c")
    sid = lax.axis_index("s")
    wid = sid * 2 + cid
    base = wid * ROWS_PER_W

    pltpu.sync_copy(pe_hbm, pe_v)

    def chunk_body(c, carry):
        off = base + c * CHUNK
        pltpu.sync_copy(x_hbm.at[pl.ds(off, CHUNK)], idx_v)
        pltpu.async_copy(table_hbm.at[idx_v], rows_v, sem).wait()

        def seq_body(k, carry2):
            def row_body(s2, carry3):
                r = k * S + s2
                rows_v[r, pl.ds(0, 16)] = (
                    rows_v[r, pl.ds(0, 16)] + pe_v[s2, pl.ds(0, 16)]
                )
                rows_v[r, pl.ds(16, 16)] = (
                    rows_v[r, pl.ds(16, 16)] + pe_v[s2, pl.ds(16, 16)]
                )
                return carry3

            return lax.fori_loop(0, S, row_body, carry2)

        lax.fori_loop(0, SEQ_PER_CHUNK, seq_body, 0)
        pltpu.sync_copy(rows_v, out_hbm.at[pl.ds(off, CHUNK)])
        return carry

    lax.fori_loop(0, NCHUNK, chunk_body, 0)


def kernel(x, table, pe):
    xf = x.reshape(B * S).astype(jnp.int32)
    pe2 = pe[0, :S, :]
    out = _emb_kernel(xf, table, pe2)
    return out.reshape(B, S, D)


# speedup vs baseline: 5.1472x; 1.2434x over previous
import functools

import jax
import jax.numpy as jnp
from jax import lax
from jax.experimental import pallas as pl
from jax.experimental.pallas import tpu as pltpu
from jax.experimental.pallas import tpu_sc as plsc

B = 4096
S = 200
D = 32
NW = 32
ROWS_PER_W = B * S // NW
CHUNK = 400
SEQ_PER_CHUNK = CHUNK // S
NCHUNK = ROWS_PER_W // CHUNK
UN = 4

_mesh = plsc.VectorSubcoreMesh(core_axis_name="c", subcore_axis_name="s")


@functools.partial(
    pl.kernel,
    mesh=_mesh,
    compiler_params=pltpu.CompilerParams(use_tc_tiling_on_sc=False),
    out_type=jax.ShapeDtypeStruct((B * S, D), jnp.float32),
    scratch_types=[
        pltpu.VMEM((ROWS_PER_W,), jnp.int32),
        pltpu.VMEM((2, CHUNK, D), jnp.float32),
        pltpu.VMEM((S, D), jnp.float32),
        pltpu.SemaphoreType.DMA,
        pltpu.SemaphoreType.DMA,
        pltpu.SemaphoreType.DMA,
        pltpu.SemaphoreType.DMA,
    ],
)
def _emb_kernel(x_hbm, table_hbm, pe_hbm, out_hbm, idx_v, rows_v, pe_v,
                sem_g0, sem_g1, sem_w0, sem_w1):
    cid = lax.axis_index("c")
    sid = lax.axis_index("s")
    wid = sid * 2 + cid
    base = wid * ROWS_PER_W
    sem_g = (sem_g0, sem_g1)
    sem_w = (sem_w0, sem_w1)

    pltpu.sync_copy(x_hbm.at[pl.ds(base, ROWS_PER_W)], idx_v)
    pltpu.sync_copy(pe_hbm, pe_v)

    def gather_start(c, buf):
        pltpu.async_copy(
            table_hbm.at[idx_v.at[pl.ds(c * CHUNK, CHUNK)]],
            rows_v.at[buf],
            sem_g[buf],
        )

    def gather_wait(buf):
        pltpu.make_async_copy(
            table_hbm.at[idx_v.at[pl.ds(0, CHUNK)]],
            rows_v.at[buf],
            sem_g[buf],
        ).wait()

    def write_start(c, buf):
        pltpu.async_copy(
            rows_v.at[buf],
            out_hbm.at[pl.ds(base + c * CHUNK, CHUNK)],
            sem_w[buf],
        )

    def write_wait(buf):
        pltpu.make_async_copy(
            rows_v.at[buf],
            out_hbm.at[pl.ds(base, CHUNK)],
            sem_w[buf],
        ).wait()

    def pe_add(buf):
        def add_body(j, carry):
            for u in range(UN):
                s2 = j * UN + u
                pe_lo = pe_v[s2, pl.ds(0, 16)]
                pe_hi = pe_v[s2, pl.ds(16, 16)]
                for k in range(SEQ_PER_CHUNK):
                    r = k * S + s2
                    rows_v[buf, r, pl.ds(0, 16)] = (
                        rows_v[buf, r, pl.ds(0, 16)] + pe_lo
                    )
                    rows_v[buf, r, pl.ds(16, 16)] = (
                        rows_v[buf, r, pl.ds(16, 16)] + pe_hi
                    )
            return carry

        lax.fori_loop(0, S // UN, add_body, 0)

    gather_start(0, 0)

    def pair_body(p, carry):
        for b in range(2):
            c = p * 2 + b
            o = 1 - b

            @pl.when(c >= 1)
            def _():
                write_wait(o)

            @pl.when(c + 1 < NCHUNK)
            def _():
                gather_start(c + 1, o)

            gather_wait(b)
            pe_add(b)
            write_start(c, b)
        return carry

    lax.fori_loop(0, NCHUNK // 2, pair_body, 0)
    write_wait((NCHUNK - 1) % 2)


def kernel(x, table, pe):
    xf = x.reshape(B * S).astype(jnp.int32)
    pe2 = pe[0, :S, :]
    out = _emb_kernel(xf, table, pe2)
    return out.reshape(B, S, D)
